# baseline (speedup 1.0000x reference)
import functools

import jax
import jax.numpy as jnp
from jax import lax
from jax.experimental import pallas as pl
from jax.experimental.pallas import tpu as pltpu
from jax.experimental.pallas import tpu_sc as plsc

N_V = 50000
N_E = 800000
D = 16
PK = 8
P_ROWS = N_E // PK
PB = 5000
N_BLK = P_ROWS // PB
EPS = 1e-5

_NC = 2
_NS = 16
_NW = _NC * _NS
_CHUNK = 125
_IDX_ROWS = N_E // _CHUNK
_RPT = _IDX_ROWS // _NW
_GRP = 8
_NGRP = _RPT // _GRP
_GRP_E = _GRP * _CHUNK
_VPT = 3200
_VPT_LAST = N_V - 15 * _VPT


def _sc_gather_body(mtab, dst2d, src2d, gd_out, gs_out,
                    idxd, idxs, buf0, buf1, gsem, wsem):
    c = lax.axis_index("c")
    s = lax.axis_index("s")
    wid = s * _NC + c
    base_row = wid * _RPT
    pltpu.sync_copy(dst2d.at[pl.ds(base_row, _RPT)], idxd)
    pltpu.sync_copy(src2d.at[pl.ds(base_row, _RPT)], idxs)

    def grp(g, _):
        @pl.when(g > 0)
        def _():
            pltpu.make_async_copy(
                buf0, gd_out.at[pl.ds(0, _GRP_E)], wsem).wait()
            pltpu.make_async_copy(
                buf1, gs_out.at[pl.ds(0, _GRP_E)], wsem).wait()

        descs = []
        for idx, buf in ((idxd, buf0), (idxs, buf1)):
            for j in range(_GRP):
                descs.append(pltpu.async_copy(
                    mtab.at[idx.at[g * _GRP + j]],
                    buf.at[pl.ds(j * _CHUNK, _CHUNK)], gsem))
        for d in descs:
            d.wait()
        off = (base_row + g * _GRP) * _CHUNK
        pltpu.async_copy(buf0, gd_out.at[pl.ds(off, _GRP_E)], wsem)
        pltpu.async_copy(buf1, gs_out.at[pl.ds(off, _GRP_E)], wsem)
        return _

    lax.fori_loop(0, _NGRP, grp, None)
    pltpu.make_async_copy(buf0, gd_out.at[pl.ds(0, _GRP_E)], wsem).wait()
    pltpu.make_async_copy(buf1, gs_out.at[pl.ds(0, _GRP_E)], wsem).wait()


def _sc_scatter_body(m_hbm, dst2d, zrows, out_hbm, idxbuf, mbuf, mbuf1,
                     accum, ssem, lsem0, lsem1):
    c = lax.axis_index("c")
    s = lax.axis_index("s")

    @pl.when(s < _NS - 1)
    def _():
        pltpu.sync_copy(zrows, accum.at[pl.ds(s * _VPT, _VPT)])

    @pl.when(s == _NS - 1)
    def _():
        pltpu.sync_copy(zrows.at[pl.ds(0, _VPT_LAST)],
                        accum.at[pl.ds(15 * _VPT, _VPT_LAST)])

    plsc.subcore_barrier()
    base_row = c * (_IDX_ROWS // _NC) + s * _RPT
    pltpu.sync_copy(dst2d.at[pl.ds(base_row, _RPT)], idxbuf)

    def _mload(g, buf, sem):
        return pltpu.async_copy(
            m_hbm.at[pl.ds((base_row + g * _GRP) * _CHUNK, _GRP_E)], buf, sem)

    def _sadds(g, buf, sem):
        descs = []
        for j in range(_GRP):
            descs.append(pltpu.async_copy(
                buf.at[pl.ds(j * _CHUNK, _CHUNK)],
                accum.at[idxbuf.at[g * _GRP + j]], sem, add=True))
        return descs

    _mload(0, mbuf, lsem0).wait()

    def grp(t, _):
        g0 = 2 * t
        ld1 = _mload(g0 + 1, mbuf1, lsem1)
        for d in _sadds(g0, mbuf, ssem):
            d.wait()
        ld2 = _mload(g0 + 2, mbuf, lsem0)
        ld1.wait()
        for d in _sadds(g0 + 1, mbuf1, ssem):
            d.wait()
        ld2.wait()
        return _

    lax.fori_loop(0, _NGRP // 2, grp, None)
    for d in _sadds(_NGRP - 1, mbuf, ssem):
        d.wait()
    plsc.subcore_barrier()

    @pl.when(s < _NS - 1)
    def _():
        pltpu.sync_copy(accum.at[pl.ds(s * _VPT, _VPT)],
                        out_hbm.at[c, pl.ds(s * _VPT, _VPT)])

    @pl.when(s == _NS - 1)
    def _():
        pltpu.sync_copy(accum.at[pl.ds(15 * _VPT, _VPT_LAST)],
                        out_hbm.at[c, pl.ds(15 * _VPT, _VPT_LAST)])


@functools.lru_cache(maxsize=None)
def _sc_kernels():
    mesh = plsc.VectorSubcoreMesh(core_axis_name="c", subcore_axis_name="s")
    sc_params = pltpu.CompilerParams(use_tc_tiling_on_sc=False)
    gather = pl.kernel(
        _sc_gather_body,
        out_type=[jax.ShapeDtypeStruct((N_E, D), jnp.float32),
                  jax.ShapeDtypeStruct((N_E, D), jnp.float32)],
        mesh=mesh,
        compiler_params=sc_params,
        scratch_types=[pltpu.VMEM((_RPT, _CHUNK), jnp.int32),
                       pltpu.VMEM((_RPT, _CHUNK), jnp.int32),
                       pltpu.VMEM((_GRP_E, D), jnp.float32),
                       pltpu.VMEM((_GRP_E, D), jnp.float32),
                       pltpu.SemaphoreType.DMA,
                       pltpu.SemaphoreType.DMA],
    )
    scatter = pl.kernel(
        _sc_scatter_body,
        out_type=jax.ShapeDtypeStruct((_NC, N_V, D), jnp.float32),
        mesh=mesh,
        compiler_params=sc_params,
        scratch_types=[pltpu.VMEM((_RPT, _CHUNK), jnp.int32),
                       pltpu.VMEM((_GRP_E, D), jnp.float32),
                       pltpu.VMEM((_GRP_E, D), jnp.float32),
                       pltpu.VMEM_SHARED((N_V, D), jnp.float32),
                       pltpu.SemaphoreType.DMA,
                       pltpu.SemaphoreType.DMA,
                       pltpu.SemaphoreType.DMA],
    )
    return gather, scatter


def _sc_gather(*args):
    return _sc_kernels()[0](*args)


def _sc_scatter(*args):
    return _sc_kernels()[1](*args)


def _edge_spec(w):
    return pl.BlockSpec((PB, w), lambda i: (i, 0))


def _full_spec(shape):
    return pl.BlockSpec(shape, lambda i: tuple(0 for _ in shape))


def _acc_stats(stats_ref, x):
    i = pl.program_id(0)

    @pl.when(i == 0)
    def _():
        stats_ref[...] = jnp.zeros_like(stats_ref)

    s = jnp.sum(x, axis=0, keepdims=True)
    q = jnp.sum(x * x, axis=0, keepdims=True)
    w = x.shape[1]
    stats_ref[...] += jnp.concatenate(
        [jnp.broadcast_to(s, (8, w)), jnp.broadcast_to(q, (8, w))], axis=0)


def _read_stats(stats, fb):
    s = jnp.sum(stats[0:8, :], axis=0, keepdims=True) / 8.0
    q = jnp.sum(stats[8:16, :], axis=0, keepdims=True) / 8.0
    sf = jnp.dot(s, fb, preferred_element_type=jnp.float32, precision=jax.lax.Precision.HIGHEST)
    qf = jnp.dot(q, fb, preferred_element_type=jnp.float32, precision=jax.lax.Precision.HIGHEST)
    mu = sf / N_E
    var = qf / N_E - mu * mu
    inv = jax.lax.rsqrt(var + EPS)
    return mu, inv


def _p1_body(gd_ref, gs_ref, hh_ref, wa_ref, wb_ref, wc_ref, b1_ref,
             stats_ref):
    x = (jnp.dot(gd_ref[...], wa_ref[...], preferred_element_type=jnp.float32)
         + jnp.dot(gs_ref[...], wb_ref[...], preferred_element_type=jnp.float32)
         + jnp.dot(hh_ref[...], wc_ref[...], preferred_element_type=jnp.float32)
         + b1_ref[...])
    _acc_stats(stats_ref, x)


def _p2_body(gd_ref, gs_ref, hh_ref, wa_ref, wb_ref, wc_ref, b1_ref,
             st_ref, fb_ref, g1_ref, be1_ref, w2_ref, b2_ref,
             va_ref, vb_ref, vb1_ref, hn_ref, st2_ref):
    x = (jnp.dot(gd_ref[...], wa_ref[...], preferred_element_type=jnp.float32)
         + jnp.dot(gs_ref[...], wb_ref[...], preferred_element_type=jnp.float32)
         + jnp.dot(hh_ref[...], wc_ref[...], preferred_element_type=jnp.float32)
         + b1_ref[...])
    mu, inv = _read_stats(st_ref[...], fb_ref[...])
    xn = (x - mu) * inv * g1_ref[...] + be1_ref[...]
    xr = jnp.maximum(xn, 0.0)
    hn = jnp.dot(xr, w2_ref[...], preferred_element_type=jnp.float32) + b2_ref[...]
    hn_ref[...] = hn
    m1 = (jnp.dot(gd_ref[...], va_ref[...], preferred_element_type=jnp.float32)
          + jnp.dot(hn, vb_ref[...], preferred_element_type=jnp.float32)
          + vb1_ref[...])
    _acc_stats(st2_ref, m1)


def _p3_body(hn_ref, gd_ref, st2_ref, fb_ref, va_ref, vb_ref, vb1_ref,
             vg1_ref, vbe1_ref, vw2_ref, vb2_ref, m_ref):
    m1 = (jnp.dot(gd_ref[...], va_ref[...], preferred_element_type=jnp.float32)
          + jnp.dot(hn_ref[...], vb_ref[...], preferred_element_type=jnp.float32)
          + vb1_ref[...])
    mu, inv = _read_stats(st2_ref[...], fb_ref[...])
    xn = (m1 - mu) * inv * vg1_ref[...] + vbe1_ref[...]
    xr = jnp.maximum(xn, 0.0)
    m_ref[...] = (jnp.dot(xr, vw2_ref[...], preferred_element_type=jnp.float32)
                  + vb2_ref[...])


def _f1_body(hh_ref, ow1_ref, ob1_ref, stats_ref):
    x = jnp.dot(hh_ref[...], ow1_ref[...],
                preferred_element_type=jnp.float32) + ob1_ref[...]
    _acc_stats(stats_ref, x)


def _f2_body(hh_ref, st_ref, fb_ref, ow1_ref, ob1_ref, og1_ref, obe1_ref,
             ows_ref, obs_ref, out_ref):
    x = jnp.dot(hh_ref[...], ow1_ref[...],
                preferred_element_type=jnp.float32) + ob1_ref[...]
    mu, inv = _read_stats(st_ref[...], fb_ref[...])
    xn = (x - mu) * inv * og1_ref[...] + obe1_ref[...]
    xr = jnp.maximum(xn, 0.0)
    d = (jnp.dot(xr, ows_ref[...], preferred_element_type=jnp.float32)
         + obs_ref[...])
    out_ref[...] = 1.0 / (1.0 + jnp.exp(d))


def _comb_body(p_ref, o_ref):
    o_ref[...] = p_ref[0] + p_ref[1]


def _bd(w):
    return jnp.kron(jnp.eye(PK, dtype=jnp.float32), w)


def _tile_row(v):
    return jnp.tile(v.reshape(1, -1), (1, PK))


def kernel(M, H, edge_index, params):
    p = params
    dst = edge_index[1].astype(jnp.int32)
    src = edge_index[0].astype(jnp.int32)
    dst2d = dst.reshape(_IDX_ROWS, _CHUNK)
    src2d = src.reshape(_IDX_ROWS, _CHUNK)
    zrows = jnp.zeros((_VPT, D), jnp.float32)

    mid_e = p['e_W1'].shape[1]
    W1 = p['e_W1']
    eWa = _bd(W1[0:D])
    eWb = _bd(W1[D:2 * D])
    eWc = _bd(W1[2 * D:3 * D])
    eb1 = _tile_row(p['e_b1'])
    eg1 = _tile_row(p['e_g1'])
    ebe1 = _tile_row(p['e_be1'])
    eW2 = _bd(p['e_W2'])
    eb2 = _tile_row(p['e_b2'])
    vWa = _bd(p['v_W1'][0:D])
    vWb = _bd(p['v_W1'][D:2 * D])
    vb1 = _tile_row(p['v_b1'])
    vg1 = _tile_row(p['v_g1'])
    vbe1 = _tile_row(p['v_be1'])
    vW2 = _bd(p['v_W2'])
    vb2 = _tile_row(p['v_b2'])
    oW1 = _bd(p['o_W1'])
    ob1 = _tile_row(p['o_b1'])
    og1 = _tile_row(p['o_g1'])
    obe1 = _tile_row(p['o_be1'])
    smat = jnp.array([[-1.0, 1.0], [1.0, -1.0]], jnp.float32)
    oWs = _bd(jnp.dot(p['o_W2'], smat))
    obs = _tile_row(jnp.dot(p['o_b2'], smat))

    we = PK * mid_e
    wd = PK * D
    fb_e = jnp.kron(jnp.ones((PK, PK), jnp.float32), jnp.eye(mid_e, dtype=jnp.float32))
    fb_d = jnp.kron(jnp.ones((PK, PK), jnp.float32), jnp.eye(D, dtype=jnp.float32))

    p1 = pl.pallas_call(
        _p1_body,
        grid=(N_BLK,),
        in_specs=[_edge_spec(wd), _edge_spec(wd), _edge_spec(wd),
                  _full_spec((wd, we)), _full_spec((wd, we)),
                  _full_spec((wd, we)), _full_spec((1, we))],
        out_specs=_full_spec((16, we)),
        out_shape=jax.ShapeDtypeStruct((16, we), jnp.float32),
    )
    p2 = pl.pallas_call(
        _p2_body,
        grid=(N_BLK,),
        in_specs=[_edge_spec(wd), _edge_spec(wd), _edge_spec(wd),
                  _full_spec((wd, we)), _full_spec((wd, we)),
                  _full_spec((wd, we)), _full_spec((1, we)),
                  _full_spec((16, we)), _full_spec((we, we)),
                  _full_spec((1, we)), _full_spec((1, we)),
                  _full_spec((we, wd)), _full_spec((1, wd)),
                  _full_spec((wd, wd)), _full_spec((wd, wd)),
                  _full_spec((1, wd))],
        out_specs=[_edge_spec(wd), _full_spec((16, wd))],
        out_shape=[jax.ShapeDtypeStruct((P_ROWS, wd), jnp.float32),
                   jax.ShapeDtypeStruct((16, wd), jnp.float32)],
    )
    p3 = pl.pallas_call(
        _p3_body,
        grid=(N_BLK,),
        in_specs=[_edge_spec(wd), _edge_spec(wd), _full_spec((16, wd)),
                  _full_spec((wd, wd)),
                  _full_spec((wd, wd)), _full_spec((wd, wd)),
                  _full_spec((1, wd)), _full_spec((1, wd)),
                  _full_spec((1, wd)), _full_spec((wd, wd)),
                  _full_spec((1, wd))],
        out_specs=_edge_spec(wd),
        out_shape=jax.ShapeDtypeStruct((P_ROWS, wd), jnp.float32),
    )
    comb = pl.pallas_call(
        _comb_body,
        grid=(1,),
        in_specs=[pl.BlockSpec((_NC, N_V // PK, wd), lambda i: (0, 0, 0))],
        out_specs=pl.BlockSpec((N_V // PK, wd), lambda i: (0, 0)),
        out_shape=jax.ShapeDtypeStruct((N_V // PK, wd), jnp.float32),
    )

    Mc = M
    Hp = H.reshape(P_ROWS, wd)
    for _ in range(20):
        gd, gs = _sc_gather(Mc, dst2d, src2d)
        gd = gd.reshape(P_ROWS, wd)
        gs = gs.reshape(P_ROWS, wd)
        st = p1(gd, gs, Hp, eWa, eWb, eWc, eb1)
        Hp, st2 = p2(gd, gs, Hp, eWa, eWb, eWc, eb1,
                     st, fb_e, eg1, ebe1, eW2, eb2, vWa, vWb, vb1)
        m = p3(Hp, gd, st2, fb_d, vWa, vWb, vb1, vg1, vbe1, vW2, vb2)
        parts = _sc_scatter(m.reshape(N_E, D), dst2d, zrows)
        Mc = comb(parts.reshape(_NC, N_V // PK, wd)).reshape(N_V, D)

    f1 = pl.pallas_call(
        _f1_body,
        grid=(N_BLK,),
        in_specs=[_edge_spec(wd), _full_spec((wd, wd)), _full_spec((1, wd))],
        out_specs=_full_spec((16, wd)),
        out_shape=jax.ShapeDtypeStruct((16, wd), jnp.float32),
    )
    f2 = pl.pallas_call(
        _f2_body,
        grid=(N_BLK,),
        in_specs=[_edge_spec(wd), _full_spec((16, wd)), _full_spec((wd, wd)),
                  _full_spec((wd, wd)), _full_spec((1, wd)),
                  _full_spec((1, wd)), _full_spec((1, wd)),
                  _full_spec((wd, 2 * PK)), _full_spec((1, 2 * PK))],
        out_specs=_edge_spec(2 * PK),
        out_shape=jax.ShapeDtypeStruct((P_ROWS, 2 * PK), jnp.float32),
    )
    stf = f1(Hp, oW1, ob1)
    out = f2(Hp, stf, fb_d, oW1, ob1, og1, obe1, oWs, obs)
    return out.reshape(N_E, 2)


# speedup vs baseline: 1.1225x; 1.1225x over previous
import functools

import jax
import jax.numpy as jnp
from jax import lax
from jax.experimental import pallas as pl
from jax.experimental.pallas import tpu as pltpu
from jax.experimental.pallas import tpu_sc as plsc

N_V = 50000
N_E = 800000
D = 16
PK = 8
P_ROWS = N_E // PK
PB = 5000
N_BLK = P_ROWS // PB
EPS = 1e-5

_NC = 2
_NS = 16
_NW = _NC * _NS
_CHUNK = 125
_IDX_ROWS = N_E // _CHUNK
_RPT = _IDX_ROWS // _NW
_GRP = 8
_NGRP = _RPT // _GRP
_GRP_E = _GRP * _CHUNK
_VPT = 3200
_VPT_LAST = N_V - 15 * _VPT


def _sc_gather_body(mtab, dst2d, src2d, gd_out, gs_out,
                    idxd, idxs, buf0, buf1, gsem, wsem):
    c = lax.axis_index("c")
    s = lax.axis_index("s")
    wid = s * _NC + c
    base_row = wid * _RPT
    pltpu.sync_copy(dst2d.at[pl.ds(base_row, _RPT)], idxd)
    pltpu.sync_copy(src2d.at[pl.ds(base_row, _RPT)], idxs)

    def grp(g, _):
        @pl.when(g > 0)
        def _():
            pltpu.make_async_copy(
                buf0, gd_out.at[pl.ds(0, _GRP_E)], wsem).wait()
            pltpu.make_async_copy(
                buf1, gs_out.at[pl.ds(0, _GRP_E)], wsem).wait()

        descs = []
        for idx, buf in ((idxd, buf0), (idxs, buf1)):
            for j in range(_GRP):
                descs.append(pltpu.async_copy(
                    mtab.at[idx.at[g * _GRP + j]],
                    buf.at[pl.ds(j * _CHUNK, _CHUNK)], gsem))
        for d in descs:
            d.wait()
        off = (base_row + g * _GRP) * _CHUNK
        pltpu.async_copy(buf0, gd_out.at[pl.ds(off, _GRP_E)], wsem)
        pltpu.async_copy(buf1, gs_out.at[pl.ds(off, _GRP_E)], wsem)
        return _

    lax.fori_loop(0, _NGRP, grp, None)
    pltpu.make_async_copy(buf0, gd_out.at[pl.ds(0, _GRP_E)], wsem).wait()
    pltpu.make_async_copy(buf1, gs_out.at[pl.ds(0, _GRP_E)], wsem).wait()


def _sc_scatter_body(m_hbm, dst2d, zrows, out_hbm, idxbuf, mbuf, mbuf1,
                     accum, ssem, lsem0, lsem1):
    c = lax.axis_index("c")
    s = lax.axis_index("s")

    @pl.when(s < _NS - 1)
    def _():
        pltpu.sync_copy(zrows, accum.at[pl.ds(s * _VPT, _VPT)])

    @pl.when(s == _NS - 1)
    def _():
        pltpu.sync_copy(zrows.at[pl.ds(0, _VPT_LAST)],
                        accum.at[pl.ds(15 * _VPT, _VPT_LAST)])

    plsc.subcore_barrier()
    base_row = c * (_IDX_ROWS // _NC) + s * _RPT
    pltpu.sync_copy(dst2d.at[pl.ds(base_row, _RPT)], idxbuf)

    def _mload(g, buf, sem):
        return pltpu.async_copy(
            m_hbm.at[pl.ds((base_row + g * _GRP) * _CHUNK, _GRP_E)], buf, sem)

    def _sadds(g, buf, sem):
        descs = []
        for j in range(_GRP):
            descs.append(pltpu.async_copy(
                buf.at[pl.ds(j * _CHUNK, _CHUNK)],
                accum.at[idxbuf.at[g * _GRP + j]], sem, add=True))
        return descs

    _mload(0, mbuf, lsem0).wait()

    def grp(t, _):
        g0 = 2 * t
        ld1 = _mload(g0 + 1, mbuf1, lsem1)
        for d in _sadds(g0, mbuf, ssem):
            d.wait()
        ld2 = _mload(g0 + 2, mbuf, lsem0)
        ld1.wait()
        for d in _sadds(g0 + 1, mbuf1, ssem):
            d.wait()
        ld2.wait()
        return _

    lax.fori_loop(0, _NGRP // 2, grp, None)
    for d in _sadds(_NGRP - 1, mbuf, ssem):
        d.wait()
    plsc.subcore_barrier()

    @pl.when(s < _NS - 1)
    def _():
        pltpu.sync_copy(accum.at[pl.ds(s * _VPT, _VPT)],
                        out_hbm.at[c, pl.ds(s * _VPT, _VPT)])

    @pl.when(s == _NS - 1)
    def _():
        pltpu.sync_copy(accum.at[pl.ds(15 * _VPT, _VPT_LAST)],
                        out_hbm.at[c, pl.ds(15 * _VPT, _VPT_LAST)])


@functools.lru_cache(maxsize=None)
def _sc_kernels():
    mesh = plsc.VectorSubcoreMesh(core_axis_name="c", subcore_axis_name="s")
    sc_params = pltpu.CompilerParams(use_tc_tiling_on_sc=False)
    gather = pl.kernel(
        _sc_gather_body,
        out_type=[jax.ShapeDtypeStruct((N_E, D), jnp.float32),
                  jax.ShapeDtypeStruct((N_E, D), jnp.float32)],
        mesh=mesh,
        compiler_params=sc_params,
        scratch_types=[pltpu.VMEM((_RPT, _CHUNK), jnp.int32),
                       pltpu.VMEM((_RPT, _CHUNK), jnp.int32),
                       pltpu.VMEM((_GRP_E, D), jnp.float32),
                       pltpu.VMEM((_GRP_E, D), jnp.float32),
                       pltpu.SemaphoreType.DMA,
                       pltpu.SemaphoreType.DMA],
    )
    scatter = pl.kernel(
        _sc_scatter_body,
        out_type=jax.ShapeDtypeStruct((_NC, N_V, D), jnp.float32),
        mesh=mesh,
        compiler_params=sc_params,
        scratch_types=[pltpu.VMEM((_RPT, _CHUNK), jnp.int32),
                       pltpu.VMEM((_GRP_E, D), jnp.float32),
                       pltpu.VMEM((_GRP_E, D), jnp.float32),
                       pltpu.VMEM_SHARED((N_V, D), jnp.float32),
                       pltpu.SemaphoreType.DMA,
                       pltpu.SemaphoreType.DMA,
                       pltpu.SemaphoreType.DMA],
    )
    return gather, scatter


def _sc_gather(*args):
    return _sc_kernels()[0](*args)


def _sc_scatter(*args):
    return _sc_kernels()[1](*args)


def _edge_spec(w):
    return pl.BlockSpec((PB, w), lambda i: (i, 0))


def _full_spec(shape):
    return pl.BlockSpec(shape, lambda i: tuple(0 for _ in shape))


def _acc_stats(stats_ref, x):
    i = pl.program_id(0)

    @pl.when(i == 0)
    def _():
        stats_ref[...] = jnp.zeros_like(stats_ref)

    s = jnp.sum(x, axis=0, keepdims=True)
    q = jnp.sum(x * x, axis=0, keepdims=True)
    w = x.shape[1]
    stats_ref[...] += jnp.concatenate(
        [jnp.broadcast_to(s, (8, w)), jnp.broadcast_to(q, (8, w))], axis=0)


def _read_stats(stats, fb):
    s = jnp.sum(stats[0:8, :], axis=0, keepdims=True) / 8.0
    q = jnp.sum(stats[8:16, :], axis=0, keepdims=True) / 8.0
    sf = jnp.dot(s, fb, preferred_element_type=jnp.float32, precision=jax.lax.Precision.HIGHEST)
    qf = jnp.dot(q, fb, preferred_element_type=jnp.float32, precision=jax.lax.Precision.HIGHEST)
    mu = sf / N_E
    var = qf / N_E - mu * mu
    inv = jax.lax.rsqrt(var + EPS)
    return mu, inv


def _p1_body(gd_ref, gs_ref, hh_ref, wa_ref, wb_ref, wc_ref, b1_ref,
             stats_ref):
    x = (jnp.dot(gd_ref[...], wa_ref[...], preferred_element_type=jnp.float32)
         + jnp.dot(gs_ref[...], wb_ref[...], preferred_element_type=jnp.float32)
         + jnp.dot(hh_ref[...], wc_ref[...], preferred_element_type=jnp.float32)
         + b1_ref[...])
    _acc_stats(stats_ref, x)


def _p1w_body(gd_ref, gs_ref, hh_ref, wa_ref, wb_ref, wc_ref, b1_ref,
              h_ref, stats_ref):
    x = (jnp.dot(gd_ref[...], wa_ref[...], preferred_element_type=jnp.float32)
         + jnp.dot(gs_ref[...], wb_ref[...], preferred_element_type=jnp.float32)
         + jnp.dot(hh_ref[...], wc_ref[...], preferred_element_type=jnp.float32)
         + b1_ref[...])
    h_ref[...] = x
    _acc_stats(stats_ref, x)


def _p2_body(h_ref, gd_ref, st_ref, fb_ref, g1_ref, be1_ref, w2_ref, b2_ref,
             va_ref, vb_ref, vb1_ref, hn_ref, st2_ref):
    mu, inv = _read_stats(st_ref[...], fb_ref[...])
    xn = (h_ref[...] - mu) * inv * g1_ref[...] + be1_ref[...]
    xr = jnp.maximum(xn, 0.0)
    hn = jnp.dot(xr, w2_ref[...], preferred_element_type=jnp.float32) + b2_ref[...]
    hn_ref[...] = hn
    m1 = (jnp.dot(gd_ref[...], va_ref[...], preferred_element_type=jnp.float32)
          + jnp.dot(hn, vb_ref[...], preferred_element_type=jnp.float32)
          + vb1_ref[...])
    _acc_stats(st2_ref, m1)


def _p3_body(hn_ref, gd_ref, st2_ref, fb_ref, va_ref, vb_ref, vb1_ref,
             vg1_ref, vbe1_ref, vw2_ref, vb2_ref, m_ref):
    m1 = (jnp.dot(gd_ref[...], va_ref[...], preferred_element_type=jnp.float32)
          + jnp.dot(hn_ref[...], vb_ref[...], preferred_element_type=jnp.float32)
          + vb1_ref[...])
    mu, inv = _read_stats(st2_ref[...], fb_ref[...])
    xn = (m1 - mu) * inv * vg1_ref[...] + vbe1_ref[...]
    xr = jnp.maximum(xn, 0.0)
    m_ref[...] = (jnp.dot(xr, vw2_ref[...], preferred_element_type=jnp.float32)
                  + vb2_ref[...])


def _f1_body(hh_ref, ow1_ref, ob1_ref, stats_ref):
    x = jnp.dot(hh_ref[...], ow1_ref[...],
                preferred_element_type=jnp.float32) + ob1_ref[...]
    _acc_stats(stats_ref, x)


def _f2_body(hh_ref, st_ref, fb_ref, ow1_ref, ob1_ref, og1_ref, obe1_ref,
             ows_ref, obs_ref, out_ref):
    x = jnp.dot(hh_ref[...], ow1_ref[...],
                preferred_element_type=jnp.float32) + ob1_ref[...]
    mu, inv = _read_stats(st_ref[...], fb_ref[...])
    xn = (x - mu) * inv * og1_ref[...] + obe1_ref[...]
    xr = jnp.maximum(xn, 0.0)
    d = (jnp.dot(xr, ows_ref[...], preferred_element_type=jnp.float32)
         + obs_ref[...])
    out_ref[...] = 1.0 / (1.0 + jnp.exp(d))


def _comb_body(p_ref, o_ref):
    o_ref[...] = p_ref[0] + p_ref[1]


def _bd(w):
    return jnp.kron(jnp.eye(PK, dtype=jnp.float32), w)


def _tile_row(v):
    return jnp.tile(v.reshape(1, -1), (1, PK))


def kernel(M, H, edge_index, params):
    p = params
    dst = edge_index[1].astype(jnp.int32)
    src = edge_index[0].astype(jnp.int32)
    dst2d = dst.reshape(_IDX_ROWS, _CHUNK)
    src2d = src.reshape(_IDX_ROWS, _CHUNK)
    zrows = jnp.zeros((_VPT, D), jnp.float32)

    mid_e = p['e_W1'].shape[1]
    W1 = p['e_W1']
    eWa = _bd(W1[0:D])
    eWb = _bd(W1[D:2 * D])
    eWc = _bd(W1[2 * D:3 * D])
    eb1 = _tile_row(p['e_b1'])
    eg1 = _tile_row(p['e_g1'])
    ebe1 = _tile_row(p['e_be1'])
    eW2 = _bd(p['e_W2'])
    eb2 = _tile_row(p['e_b2'])
    vWa = _bd(p['v_W1'][0:D])
    vWb = _bd(p['v_W1'][D:2 * D])
    vb1 = _tile_row(p['v_b1'])
    vg1 = _tile_row(p['v_g1'])
    vbe1 = _tile_row(p['v_be1'])
    vW2 = _bd(p['v_W2'])
    vb2 = _tile_row(p['v_b2'])
    oW1 = _bd(p['o_W1'])
    ob1 = _tile_row(p['o_b1'])
    og1 = _tile_row(p['o_g1'])
    obe1 = _tile_row(p['o_be1'])
    smat = jnp.array([[-1.0, 1.0], [1.0, -1.0]], jnp.float32)
    oWs = _bd(jnp.dot(p['o_W2'], smat))
    obs = _tile_row(jnp.dot(p['o_b2'], smat))

    we = PK * mid_e
    wd = PK * D
    fb_e = jnp.kron(jnp.ones((PK, PK), jnp.float32), jnp.eye(mid_e, dtype=jnp.float32))
    fb_d = jnp.kron(jnp.ones((PK, PK), jnp.float32), jnp.eye(D, dtype=jnp.float32))

    p1 = pl.pallas_call(
        _p1w_body,
        grid=(N_BLK,),
        in_specs=[_edge_spec(wd), _edge_spec(wd), _edge_spec(wd),
                  _full_spec((wd, we)), _full_spec((wd, we)),
                  _full_spec((wd, we)), _full_spec((1, we))],
        out_specs=[_edge_spec(we), _full_spec((16, we))],
        out_shape=[jax.ShapeDtypeStruct((P_ROWS, we), jnp.float32),
                   jax.ShapeDtypeStruct((16, we), jnp.float32)],
    )
    p2 = pl.pallas_call(
        _p2_body,
        grid=(N_BLK,),
        in_specs=[_edge_spec(we), _edge_spec(wd), _full_spec((16, we)),
                  _full_spec((we, we)),
                  _full_spec((1, we)), _full_spec((1, we)),
                  _full_spec((we, wd)), _full_spec((1, wd)),
                  _full_spec((wd, wd)), _full_spec((wd, wd)),
                  _full_spec((1, wd))],
        out_specs=[_edge_spec(wd), _full_spec((16, wd))],
        out_shape=[jax.ShapeDtypeStruct((P_ROWS, wd), jnp.float32),
                   jax.ShapeDtypeStruct((16, wd), jnp.float32)],
    )
    p3 = pl.pallas_call(
        _p3_body,
        grid=(N_BLK,),
        in_specs=[_edge_spec(wd), _edge_spec(wd), _full_spec((16, wd)),
                  _full_spec((wd, wd)),
                  _full_spec((wd, wd)), _full_spec((wd, wd)),
                  _full_spec((1, wd)), _full_spec((1, wd)),
                  _full_spec((1, wd)), _full_spec((wd, wd)),
                  _full_spec((1, wd))],
        out_specs=_edge_spec(wd),
        out_shape=jax.ShapeDtypeStruct((P_ROWS, wd), jnp.float32),
    )
    comb = pl.pallas_call(
        _comb_body,
        grid=(1,),
        in_specs=[pl.BlockSpec((_NC, N_V // PK, wd), lambda i: (0, 0, 0))],
        out_specs=pl.BlockSpec((N_V // PK, wd), lambda i: (0, 0)),
        out_shape=jax.ShapeDtypeStruct((N_V // PK, wd), jnp.float32),
    )

    Mc = M
    Hp = H.reshape(P_ROWS, wd)
    for _ in range(20):
        gd, gs = _sc_gather(Mc, dst2d, src2d)
        gd = gd.reshape(P_ROWS, wd)
        gs = gs.reshape(P_ROWS, wd)
        h, st = p1(gd, gs, Hp, eWa, eWb, eWc, eb1)
        Hp, st2 = p2(h, gd, st, fb_e, eg1, ebe1, eW2, eb2, vWa, vWb, vb1)
        m = p3(Hp, gd, st2, fb_d, vWa, vWb, vb1, vg1, vbe1, vW2, vb2)
        parts = _sc_scatter(m.reshape(N_E, D), dst2d, zrows)
        Mc = comb(parts.reshape(_NC, N_V // PK, wd)).reshape(N_V, D)

    f1 = pl.pallas_call(
        _f1_body,
        grid=(N_BLK,),
        in_specs=[_edge_spec(wd), _full_spec((wd, wd)), _full_spec((1, wd))],
        out_specs=_full_spec((16, wd)),
        out_shape=jax.ShapeDtypeStruct((16, wd), jnp.float32),
    )
    f2 = pl.pallas_call(
        _f2_body,
        grid=(N_BLK,),
        in_specs=[_edge_spec(wd), _full_spec((16, wd)), _full_spec((wd, wd)),
                  _full_spec((wd, wd)), _full_spec((1, wd)),
                  _full_spec((1, wd)), _full_spec((1, wd)),
                  _full_spec((wd, 2 * PK)), _full_spec((1, 2 * PK))],
        out_specs=_edge_spec(2 * PK),
        out_shape=jax.ShapeDtypeStruct((P_ROWS, 2 * PK), jnp.float32),
    )
    stf = f1(Hp, oW1, ob1)
    out = f2(Hp, stf, fb_d, oW1, ob1, og1, obe1, oWs, obs)
    return out.reshape(N_E, 2)


# speedup vs baseline: 1.1229x; 1.0003x over previous
import functools

import jax
import jax.numpy as jnp
from jax import lax
from jax.experimental import pallas as pl
from jax.experimental.pallas import tpu as pltpu
from jax.experimental.pallas import tpu_sc as plsc

N_V = 50000
N_E = 800000
D = 16
PK = 8
P_ROWS = N_E // PK
PB = 5000
N_BLK = P_ROWS // PB
EPS = 1e-5

_NC = 2
_NS = 16
_NW = _NC * _NS
_CHUNK = 125
_IDX_ROWS = N_E // _CHUNK
_RPT = _IDX_ROWS // _NW
_GRP = 8
_NGRP = _RPT // _GRP
_GRP_E = _GRP * _CHUNK
_VPT = 3200
_VPT_LAST = N_V - 15 * _VPT


def _sc_gather_body(mtab, dst2d, src2d, gd_out, gs_out,
                    idxd, idxs, buf0, buf1, gsem, wsem):
    c = lax.axis_index("c")
    s = lax.axis_index("s")
    wid = s * _NC + c
    base_row = wid * _RPT
    pltpu.sync_copy(dst2d.at[pl.ds(base_row, _RPT)], idxd)
    pltpu.sync_copy(src2d.at[pl.ds(base_row, _RPT)], idxs)

    def grp(g, _):
        @pl.when(g > 0)
        def _():
            pltpu.make_async_copy(
                buf0, gd_out.at[pl.ds(0, _GRP_E)], wsem).wait()
            pltpu.make_async_copy(
                buf1, gs_out.at[pl.ds(0, _GRP_E)], wsem).wait()

        descs = []
        for idx, buf in ((idxd, buf0), (idxs, buf1)):
            for j in range(_GRP):
                descs.append(pltpu.async_copy(
                    mtab.at[idx.at[g * _GRP + j]],
                    buf.at[pl.ds(j * _CHUNK, _CHUNK)], gsem))
        for d in descs:
            d.wait()
        off = (base_row + g * _GRP) * _CHUNK
        pltpu.async_copy(buf0, gd_out.at[pl.ds(off, _GRP_E)], wsem)
        pltpu.async_copy(buf1, gs_out.at[pl.ds(off, _GRP_E)], wsem)
        return _

    lax.fori_loop(0, _NGRP, grp, None)
    pltpu.make_async_copy(buf0, gd_out.at[pl.ds(0, _GRP_E)], wsem).wait()
    pltpu.make_async_copy(buf1, gs_out.at[pl.ds(0, _GRP_E)], wsem).wait()


def _sc_scatter_body(m_hbm, dst2d, zrows, out_hbm, idxbuf, mbuf, mbuf1,
                     accum, ssem, lsem0, lsem1):
    c = lax.axis_index("c")
    s = lax.axis_index("s")

    @pl.when(s < _NS - 1)
    def _():
        pltpu.sync_copy(zrows, accum.at[pl.ds(s * _VPT, _VPT)])

    @pl.when(s == _NS - 1)
    def _():
        pltpu.sync_copy(zrows.at[pl.ds(0, _VPT_LAST)],
                        accum.at[pl.ds(15 * _VPT, _VPT_LAST)])

    plsc.subcore_barrier()
    base_row = c * (_IDX_ROWS // _NC) + s * _RPT
    pltpu.sync_copy(dst2d.at[pl.ds(base_row, _RPT)], idxbuf)

    def _mload(g, buf, sem):
        return pltpu.async_copy(
            m_hbm.at[pl.ds((base_row + g * _GRP) * _CHUNK, _GRP_E)], buf, sem)

    def _sadds(g, buf, sem):
        descs = []
        for j in range(_GRP):
            descs.append(pltpu.async_copy(
                buf.at[pl.ds(j * _CHUNK, _CHUNK)],
                accum.at[idxbuf.at[g * _GRP + j]], sem, add=True))
        return descs

    _mload(0, mbuf, lsem0).wait()

    def grp(t, _):
        g0 = 2 * t
        ld1 = _mload(g0 + 1, mbuf1, lsem1)
        for d in _sadds(g0, mbuf, ssem):
            d.wait()
        ld2 = _mload(g0 + 2, mbuf, lsem0)
        ld1.wait()
        for d in _sadds(g0 + 1, mbuf1, ssem):
            d.wait()
        ld2.wait()
        return _

    lax.fori_loop(0, _NGRP // 2, grp, None)
    for d in _sadds(_NGRP - 1, mbuf, ssem):
        d.wait()
    plsc.subcore_barrier()

    @pl.when(s < _NS - 1)
    def _():
        pltpu.sync_copy(accum.at[pl.ds(s * _VPT, _VPT)],
                        out_hbm.at[c, pl.ds(s * _VPT, _VPT)])

    @pl.when(s == _NS - 1)
    def _():
        pltpu.sync_copy(accum.at[pl.ds(15 * _VPT, _VPT_LAST)],
                        out_hbm.at[c, pl.ds(15 * _VPT, _VPT_LAST)])


@functools.lru_cache(maxsize=None)
def _sc_kernels():
    mesh = plsc.VectorSubcoreMesh(core_axis_name="c", subcore_axis_name="s")
    sc_params = pltpu.CompilerParams(use_tc_tiling_on_sc=False)
    gather = pl.kernel(
        _sc_gather_body,
        out_type=[jax.ShapeDtypeStruct((N_E, D), jnp.float32),
                  jax.ShapeDtypeStruct((N_E, D), jnp.float32)],
        mesh=mesh,
        compiler_params=sc_params,
        scratch_types=[pltpu.VMEM((_RPT, _CHUNK), jnp.int32),
                       pltpu.VMEM((_RPT, _CHUNK), jnp.int32),
                       pltpu.VMEM((_GRP_E, D), jnp.float32),
                       pltpu.VMEM((_GRP_E, D), jnp.float32),
                       pltpu.SemaphoreType.DMA,
                       pltpu.SemaphoreType.DMA],
    )
    scatter = pl.kernel(
        _sc_scatter_body,
        out_type=jax.ShapeDtypeStruct((_NC, N_V, D), jnp.float32),
        mesh=mesh,
        compiler_params=sc_params,
        scratch_types=[pltpu.VMEM((_RPT, _CHUNK), jnp.int32),
                       pltpu.VMEM((_GRP_E, D), jnp.float32),
                       pltpu.VMEM((_GRP_E, D), jnp.float32),
                       pltpu.VMEM_SHARED((N_V, D), jnp.float32),
                       pltpu.SemaphoreType.DMA,
                       pltpu.SemaphoreType.DMA,
                       pltpu.SemaphoreType.DMA],
    )
    return gather, scatter


def _sc_gather(*args):
    return _sc_kernels()[0](*args)


def _sc_scatter(*args):
    return _sc_kernels()[1](*args)


def _edge_spec(w):
    return pl.BlockSpec((PB, w), lambda i: (i, 0))


def _full_spec(shape):
    return pl.BlockSpec(shape, lambda i: tuple(0 for _ in shape))


def _acc_stats(stats_ref, x):
    i = pl.program_id(0)

    @pl.when(i == 0)
    def _():
        stats_ref[...] = jnp.zeros_like(stats_ref)

    s = jnp.sum(x, axis=0, keepdims=True)
    q = jnp.sum(x * x, axis=0, keepdims=True)
    w = x.shape[1]
    stats_ref[...] += jnp.concatenate(
        [jnp.broadcast_to(s, (8, w)), jnp.broadcast_to(q, (8, w))], axis=0)


def _read_stats(stats, fb):
    s = jnp.sum(stats[0:8, :], axis=0, keepdims=True) / 8.0
    q = jnp.sum(stats[8:16, :], axis=0, keepdims=True) / 8.0
    sf = jnp.dot(s, fb, preferred_element_type=jnp.float32, precision=jax.lax.Precision.HIGHEST)
    qf = jnp.dot(q, fb, preferred_element_type=jnp.float32, precision=jax.lax.Precision.HIGHEST)
    mu = sf / N_E
    var = qf / N_E - mu * mu
    inv = jax.lax.rsqrt(var + EPS)
    return mu, inv


def _p1_body(gd_ref, gs_ref, hh_ref, wa_ref, wb_ref, wc_ref, b1_ref,
             stats_ref):
    x = (jnp.dot(gd_ref[...], wa_ref[...], preferred_element_type=jnp.float32)
         + jnp.dot(gs_ref[...], wb_ref[...], preferred_element_type=jnp.float32)
         + jnp.dot(hh_ref[...], wc_ref[...], preferred_element_type=jnp.float32)
         + b1_ref[...])
    _acc_stats(stats_ref, x)


def _p1w_body(gd_ref, gs_ref, hh_ref, wa_ref, wb_ref, wc_ref, b1_ref,
              h_ref, stats_ref):
    x = (jnp.dot(gd_ref[...], wa_ref[...], preferred_element_type=jnp.float32)
         + jnp.dot(gs_ref[...], wb_ref[...], preferred_element_type=jnp.float32)
         + jnp.dot(hh_ref[...], wc_ref[...], preferred_element_type=jnp.float32)
         + b1_ref[...])
    h_ref[...] = x
    _acc_stats(stats_ref, x)


def _p2_body(h_ref, gd_ref, st_ref, fb_ref, g1_ref, be1_ref, w2_ref, b2_ref,
             va_ref, vb_ref, vb1_ref, hn_ref, st2_ref):
    mu, inv = _read_stats(st_ref[...], fb_ref[...])
    xn = (h_ref[...] - mu) * inv * g1_ref[...] + be1_ref[...]
    xr = jnp.maximum(xn, 0.0)
    hn = jnp.dot(xr, w2_ref[...], preferred_element_type=jnp.float32) + b2_ref[...]
    hn_ref[...] = hn
    m1 = (jnp.dot(gd_ref[...], va_ref[...], preferred_element_type=jnp.float32)
          + jnp.dot(hn, vb_ref[...], preferred_element_type=jnp.float32)
          + vb1_ref[...])
    _acc_stats(st2_ref, m1)


def _p3_body(hn_ref, gd_ref, st2_ref, fb_ref, va_ref, vb_ref, vb1_ref,
             vg1_ref, vbe1_ref, vw2_ref, vb2_ref, m_ref):
    m1 = (jnp.dot(gd_ref[...], va_ref[...], preferred_element_type=jnp.float32)
          + jnp.dot(hn_ref[...], vb_ref[...], preferred_element_type=jnp.float32)
          + vb1_ref[...])
    mu, inv = _read_stats(st2_ref[...], fb_ref[...])
    xn = (m1 - mu) * inv * vg1_ref[...] + vbe1_ref[...]
    xr = jnp.maximum(xn, 0.0)
    m_ref[...] = (jnp.dot(xr, vw2_ref[...], preferred_element_type=jnp.float32)
                  + vb2_ref[...])


def _f1_body(hh_ref, ow1_ref, ob1_ref, stats_ref):
    x = jnp.dot(hh_ref[...], ow1_ref[...],
                preferred_element_type=jnp.float32) + ob1_ref[...]
    _acc_stats(stats_ref, x)


def _f2_body(hh_ref, st_ref, fb_ref, ow1_ref, ob1_ref, og1_ref, obe1_ref,
             ows_ref, obs_ref, out_ref):
    x = jnp.dot(hh_ref[...], ow1_ref[...],
                preferred_element_type=jnp.float32) + ob1_ref[...]
    mu, inv = _read_stats(st_ref[...], fb_ref[...])
    xn = (x - mu) * inv * og1_ref[...] + obe1_ref[...]
    xr = jnp.maximum(xn, 0.0)
    d = (jnp.dot(xr, ows_ref[...], preferred_element_type=jnp.float32)
         + obs_ref[...])
    out_ref[...] = 1.0 / (1.0 + jnp.exp(d))


def _comb_body(p_ref, o_ref):
    o_ref[...] = p_ref[0] + p_ref[1]


def _bd(w):
    return jnp.kron(jnp.eye(PK, dtype=jnp.float32), w)


def _tile_row(v):
    return jnp.tile(v.reshape(1, -1), (1, PK))


def kernel(M, H, edge_index, params):
    p = params
    dst = edge_index[1].astype(jnp.int32)
    src = edge_index[0].astype(jnp.int32)
    dst2d = dst.reshape(_IDX_ROWS, _CHUNK)
    src2d = src.reshape(_IDX_ROWS, _CHUNK)
    zrows = jnp.zeros((_VPT, D), jnp.float32)

    mid_e = p['e_W1'].shape[1]
    W1 = p['e_W1']
    eWa = _bd(W1[0:D])
    eWb = _bd(W1[D:2 * D])
    eWc = _bd(W1[2 * D:3 * D])
    eb1 = _tile_row(p['e_b1'])
    eg1 = _tile_row(p['e_g1'])
    ebe1 = _tile_row(p['e_be1'])
    eW2 = _bd(p['e_W2'])
    eb2 = _tile_row(p['e_b2'])
    vWa = _bd(p['v_W1'][0:D])
    vWb = _bd(p['v_W1'][D:2 * D])
    vb1 = _tile_row(p['v_b1'])
    vg1 = _tile_row(p['v_g1'])
    vbe1 = _tile_row(p['v_be1'])
    vW2 = _bd(p['v_W2'])
    vb2 = _tile_row(p['v_b2'])
    oW1 = _bd(p['o_W1'])
    ob1 = _tile_row(p['o_b1'])
    og1 = _tile_row(p['o_g1'])
    obe1 = _tile_row(p['o_be1'])
    smat = jnp.array([[-1.0, 1.0], [1.0, -1.0]], jnp.float32)
    oWs = _bd(jnp.dot(p['o_W2'], smat))
    obs = _tile_row(jnp.dot(p['o_b2'], smat))

    we = PK * mid_e
    wd = PK * D
    fb_e = jnp.kron(jnp.ones((PK, PK), jnp.float32), jnp.eye(mid_e, dtype=jnp.float32))
    fb_d = jnp.kron(jnp.ones((PK, PK), jnp.float32), jnp.eye(D, dtype=jnp.float32))

    p1 = pl.pallas_call(
        _p1w_body,
        grid=(N_BLK,),
        in_specs=[_edge_spec(wd), _edge_spec(wd), _edge_spec(wd),
                  _full_spec((wd, we)), _full_spec((wd, we)),
                  _full_spec((wd, we)), _full_spec((1, we))],
        out_specs=[_edge_spec(we), _full_spec((16, we))],
        out_shape=[jax.ShapeDtypeStruct((P_ROWS, we), jnp.float32),
                   jax.ShapeDtypeStruct((16, we), jnp.float32)],
    )
    p2 = pl.pallas_call(
        _p2_body,
        grid=(N_BLK,),
        in_specs=[_edge_spec(we), _edge_spec(wd), _full_spec((16, we)),
                  _full_spec((we, we)),
                  _full_spec((1, we)), _full_spec((1, we)),
                  _full_spec((we, wd)), _full_spec((1, wd)),
                  _full_spec((wd, wd)), _full_spec((wd, wd)),
                  _full_spec((1, wd))],
        out_specs=[_edge_spec(wd), _full_spec((16, wd))],
        out_shape=[jax.ShapeDtypeStruct((P_ROWS, wd), jnp.float32),
                   jax.ShapeDtypeStruct((16, wd), jnp.float32)],
    )
    p3 = pl.pallas_call(
        _p3_body,
        grid=(N_BLK,),
        in_specs=[_edge_spec(wd), _edge_spec(wd), _full_spec((16, wd)),
                  _full_spec((wd, wd)),
                  _full_spec((wd, wd)), _full_spec((wd, wd)),
                  _full_spec((1, wd)), _full_spec((1, wd)),
                  _full_spec((1, wd)), _full_spec((wd, wd)),
                  _full_spec((1, wd))],
        out_specs=_edge_spec(wd),
        out_shape=jax.ShapeDtypeStruct((P_ROWS, wd), jnp.float32),
    )
    comb = pl.pallas_call(
        _comb_body,
        grid=(1,),
        in_specs=[pl.BlockSpec((_NC, N_V // PK, wd), lambda i: (0, 0, 0))],
        out_specs=pl.BlockSpec((N_V // PK, wd), lambda i: (0, 0)),
        out_shape=jax.ShapeDtypeStruct((N_V // PK, wd), jnp.float32),
    )

    Mc = M
    Hp = H.reshape(P_ROWS, wd)
    for layer in range(20):
        gd, gs = _sc_gather(Mc, dst2d, src2d)
        gd = gd.reshape(P_ROWS, wd)
        gs = gs.reshape(P_ROWS, wd)
        h, st = p1(gd, gs, Hp, eWa, eWb, eWc, eb1)
        Hp, st2 = p2(h, gd, st, fb_e, eg1, ebe1, eW2, eb2, vWa, vWb, vb1)
        if layer < 19:
            m = p3(Hp, gd, st2, fb_d, vWa, vWb, vb1, vg1, vbe1, vW2, vb2)
            parts = _sc_scatter(m.reshape(N_E, D), dst2d, zrows)
            Mc = comb(parts.reshape(_NC, N_V // PK, wd)).reshape(N_V, D)

    f1 = pl.pallas_call(
        _f1_body,
        grid=(N_BLK,),
        in_specs=[_edge_spec(wd), _full_spec((wd, wd)), _full_spec((1, wd))],
        out_specs=_full_spec((16, wd)),
        out_shape=jax.ShapeDtypeStruct((16, wd), jnp.float32),
    )
    f2 = pl.pallas_call(
        _f2_body,
        grid=(N_BLK,),
        in_specs=[_edge_spec(wd), _full_spec((16, wd)), _full_spec((wd, wd)),
                  _full_spec((wd, wd)), _full_spec((1, wd)),
                  _full_spec((1, wd)), _full_spec((1, wd)),
                  _full_spec((wd, 2 * PK)), _full_spec((1, 2 * PK))],
        out_specs=_edge_spec(2 * PK),
        out_shape=jax.ShapeDtypeStruct((P_ROWS, 2 * PK), jnp.float32),
    )
    stf = f1(Hp, oW1, ob1)
    out = f2(Hp, stf, fb_d, oW1, ob1, og1, obe1, oWs, obs)
    return out.reshape(N_E, 2)
